# Initial kernel scaffold; baseline (speedup 1.0000x reference)
import jax
import jax.numpy as jnp
from jax.experimental import pallas as pl


def kernel(x, edge_index, W_self0, W_neigh0, b0, W_self1, W_neigh1, b1, W_self2, W_neigh2, b2):
    raise NotImplementedError("write your pallas kernel here")


# speedup vs baseline: 4.9690x; 4.9690x over previous
import functools

import jax
import jax.numpy as jnp
from jax import lax
from jax.experimental import pallas as pl
from jax.experimental.pallas import tpu as pltpu
from jax.experimental.pallas import tpu_sc as plsc

_N = 10000
_E = 320000
_NC = 2
_NS = 16
_NW = _NC * _NS
_EPW = _E // _NW
_C = 80
_NCHUNK = _EPW // _C
_NP = 10240
_RPT = _NP // _NS


def _sc_scatter(d_feat, with_deg):
    mesh = plsc.VectorSubcoreMesh(core_axis_name="c", subcore_axis_name="s")
    out_type = [jax.ShapeDtypeStruct((_NC * _NP, d_feat), jnp.float32)]
    scratch = [
        pltpu.VMEM((_C,), jnp.int32),
        pltpu.VMEM((_C,), jnp.int32),
        pltpu.VMEM((_C, d_feat), jnp.float32),
        pltpu.VMEM_SHARED((_NP, d_feat), jnp.float32),
        pltpu.SemaphoreType.DMA,
    ]
    if with_deg:
        out_type.append(jax.ShapeDtypeStruct((_NC * _NP,), jnp.float32))
        scratch += [
            pltpu.VMEM((_NP,), jnp.float32),
            pltpu.VMEM((_RPT,), jnp.float32),
            pltpu.VMEM((_RPT,), jnp.float32),
            pltpu.VMEM_SHARED((_NS, _NP), jnp.float32),
        ]

    @functools.partial(
        pl.kernel, mesh=mesh, out_type=tuple(out_type),
        scratch_types=scratch,
        compiler_params=pltpu.CompilerParams(needs_layout_passes=False),
    )
    def k(*refs):
        if with_deg:
            (t_hbm, src_hbm, dst_hbm, z_hbm, z1_hbm,
             p_hbm, pd_hbm, src_v, dst_v, rows_v, acc, sem,
             deg_v, dsum_v, tmp_v, sdeg) = refs
        else:
            (t_hbm, src_hbm, dst_hbm, z_hbm,
             p_hbm, src_v, dst_v, rows_v, acc, sem) = refs
        cid = lax.axis_index("c")
        sid = lax.axis_index("s")
        r0 = sid * _RPT

        _chunks = [(k * _C, _C) for k in range(_RPT // _C)]

        def bounce(src_r, s_off, dst_r, d_off, buf):
            for o, sz in _chunks:
                pltpu.sync_copy(src_r.at[pl.ds(s_off + o, sz)],
                                buf.at[pl.ds(0, sz)])
                pltpu.sync_copy(buf.at[pl.ds(0, sz)],
                                dst_r.at[pl.ds(d_off + o, sz)])

        bounce(z_hbm, r0, acc, r0, rows_v)
        if with_deg:
            pltpu.sync_copy(z1_hbm, deg_v)
        plsc.subcore_barrier()

        base = cid * (_E // _NC) + sid * _EPW

        def body(j, carry):
            e0 = base + j * _C
            pltpu.sync_copy(src_hbm.at[pl.ds(e0, _C)], src_v)
            pltpu.sync_copy(dst_hbm.at[pl.ds(e0, _C)], dst_v)
            pltpu.async_copy(t_hbm.at[src_v], rows_v, sem).wait()
            pltpu.sync_copy(rows_v, acc.at[dst_v], add=True)
            if with_deg:
                for i in range(_C // 16):
                    d16 = dst_v[pl.ds(i * 16, 16)]
                    plsc.addupdate_scatter(
                        deg_v, [d16], jnp.full((16,), 1.0, jnp.float32))
            return carry

        lax.fori_loop(0, _NCHUNK, body, 0)
        if with_deg:
            pltpu.sync_copy(deg_v, sdeg.at[sid])
        plsc.subcore_barrier()

        o0 = cid * _NP + r0
        bounce(acc, r0, p_hbm, o0, rows_v)
        if with_deg:
            pltpu.sync_copy(sdeg.at[0, pl.ds(r0, _RPT)], dsum_v)

            def rbody(t, carry):
                pltpu.sync_copy(sdeg.at[t, pl.ds(r0, _RPT)], tmp_v)
                for i in range(_RPT // 16):
                    plsc.addupdate(dsum_v.at[pl.ds(i * 16, 16)],
                                   tmp_v[pl.ds(i * 16, 16)])
                return carry

            lax.fori_loop(1, _NS, rbody, 0)
            pltpu.sync_copy(dsum_v, pd_hbm.at[pl.ds(o0, _RPT)])

    return k


_BR = 1000


def _tc_transform_body(x_ref, w_ref, o_ref):
    o_ref[...] = jnp.dot(x_ref[...], w_ref[...],
                         preferred_element_type=jnp.float32)


def _tc_transform(x, w):
    d_in, d_out = w.shape
    return pl.pallas_call(
        _tc_transform_body,
        grid=(_N // _BR,),
        in_specs=[pl.BlockSpec((_BR, d_in), lambda i: (i, 0)),
                  pl.BlockSpec((d_in, d_out), lambda i: (0, 0))],
        out_specs=pl.BlockSpec((_BR, d_out), lambda i: (i, 0)),
        out_shape=jax.ShapeDtypeStruct((_N, d_out), jnp.float32),
    )(x, w)


def _combine(h, w_self, b, p0, p1, d0, d1):
    inv = 1.0 / jnp.maximum(d0 + d1, 1.0)
    return (jnp.dot(h, w_self, preferred_element_type=jnp.float32)
            + (p0 + p1) * inv + b)


def _tc_mid_body(h_ref, p0_ref, p1_ref, d0_ref, d1_ref, ws_ref, b_ref,
                 wn_ref, hn_ref, tn_ref):
    z = _combine(h_ref[...], ws_ref[...], b_ref[...], p0_ref[...],
                 p1_ref[...], d0_ref[...], d1_ref[...])
    hn = jnp.where(z > 0, z, jnp.exp(jnp.minimum(z, 0.0)) - 1.0)
    hn_ref[...] = hn
    tn_ref[...] = jnp.dot(hn, wn_ref[...], preferred_element_type=jnp.float32)


def _tc_mid(h, p0, p1, d0, d1, w_self, b, w_next):
    d = h.shape[1]
    dn = w_next.shape[1]
    return pl.pallas_call(
        _tc_mid_body,
        grid=(_N // _BR,),
        in_specs=[pl.BlockSpec((_BR, d), lambda i: (i, 0)),
                  pl.BlockSpec((_BR, d), lambda i: (i, 0)),
                  pl.BlockSpec((_BR, d), lambda i: (i, 0)),
                  pl.BlockSpec((_BR, 1), lambda i: (i, 0)),
                  pl.BlockSpec((_BR, 1), lambda i: (i, 0)),
                  pl.BlockSpec((d, d), lambda i: (0, 0)),
                  pl.BlockSpec((1, d), lambda i: (0, 0)),
                  pl.BlockSpec((d, dn), lambda i: (0, 0))],
        out_specs=[pl.BlockSpec((_BR, d), lambda i: (i, 0)),
                   pl.BlockSpec((_BR, dn), lambda i: (i, 0))],
        out_shape=[jax.ShapeDtypeStruct((_N, d), jnp.float32),
                   jax.ShapeDtypeStruct((_N, dn), jnp.float32)],
    )(h, p0, p1, d0, d1, w_self, b.reshape(1, d), w_next)


def _tc_last_body(h_ref, p0_ref, p1_ref, d0_ref, d1_ref, ws_ref, b_ref,
                  o_ref):
    o_ref[...] = _combine(h_ref[...], ws_ref[...], b_ref[...], p0_ref[...],
                          p1_ref[...], d0_ref[...], d1_ref[...])


def _tc_last(h, p0, p1, d0, d1, w_self, b):
    d = h.shape[1]
    dn = w_self.shape[1]
    return pl.pallas_call(
        _tc_last_body,
        grid=(_N // _BR,),
        in_specs=[pl.BlockSpec((_BR, d), lambda i: (i, 0)),
                  pl.BlockSpec((_BR, dn), lambda i: (i, 0)),
                  pl.BlockSpec((_BR, dn), lambda i: (i, 0)),
                  pl.BlockSpec((_BR, 1), lambda i: (i, 0)),
                  pl.BlockSpec((_BR, 1), lambda i: (i, 0)),
                  pl.BlockSpec((d, dn), lambda i: (0, 0)),
                  pl.BlockSpec((1, dn), lambda i: (0, 0))],
        out_specs=pl.BlockSpec((_BR, dn), lambda i: (i, 0)),
        out_shape=jax.ShapeDtypeStruct((_N, dn), jnp.float32),
    )(h, p0, p1, d0, d1, w_self, b.reshape(1, dn))


def kernel(x, edge_index,
           W_self0, W_neigh0, b0,
           W_self1, W_neigh1, b1,
           W_self2, W_neigh2, b2):
    src = edge_index[0].astype(jnp.int32)
    dst = edge_index[1].astype(jnp.int32)
    z128 = jnp.zeros((_NP, 128), jnp.float32)
    z1 = jnp.zeros((_NP,), jnp.float32)

    sc128_deg = _sc_scatter(128, True)
    sc128 = _sc_scatter(128, False)

    t0 = _tc_transform(x, W_neigh0)
    p, pd = sc128_deg(t0, src, dst, z128, z1)
    d0 = pd[:_N, None]
    d1 = pd[_NP:_NP + _N, None]
    h1, t1 = _tc_mid(x, p[:_N], p[_NP:_NP + _N], d0, d1, W_self0, b0,
                     W_neigh1)
    (p,) = sc128(t1, src, dst, z128)
    w2p = jnp.pad(W_neigh2, ((0, 0), (0, 64)))
    h2, t2 = _tc_mid(h1, p[:_N], p[_NP:_NP + _N], d0, d1, W_self1, b1, w2p)
    (p,) = sc128(t2, src, dst, z128)
    return _tc_last(h2, p[:_N, :64], p[_NP:_NP + _N, :64], d0, d1,
                    W_self2, b2)


# speedup vs baseline: 9.1239x; 1.8362x over previous
import functools

import jax
import jax.numpy as jnp
from jax import lax
from jax.experimental import pallas as pl
from jax.experimental.pallas import tpu as pltpu
from jax.experimental.pallas import tpu_sc as plsc

_N = 10000
_E = 320000
_NC = 2
_NS = 16
_NW = _NC * _NS
_EPW = _E // _NW
_C = 80
_NCHUNK = _EPW // _C
_NP = 10240
_RPT = _NP // _NS


def _sc_scatter(d_feat, with_deg):
    mesh = plsc.VectorSubcoreMesh(core_axis_name="c", subcore_axis_name="s")
    out_type = [jax.ShapeDtypeStruct((_NC * _NP, d_feat), jnp.float32)]
    scratch = [
        pltpu.VMEM((_C,), jnp.int32),
        pltpu.VMEM((_C,), jnp.int32),
        pltpu.VMEM((_C,), jnp.int32),
        pltpu.VMEM((_C,), jnp.int32),
        pltpu.VMEM((_C, d_feat), jnp.float32),
        pltpu.VMEM((_C, d_feat), jnp.float32),
        pltpu.VMEM_SHARED((_NP, d_feat), jnp.float32),
        pltpu.SemaphoreType.DMA,
        pltpu.SemaphoreType.DMA,
        pltpu.SemaphoreType.DMA,
    ]
    if with_deg:
        out_type.append(jax.ShapeDtypeStruct((_NC * _NP,), jnp.float32))
        scratch += [
            pltpu.VMEM((_NP,), jnp.float32),
            pltpu.VMEM((_RPT,), jnp.float32),
            pltpu.VMEM((_RPT,), jnp.float32),
            pltpu.VMEM_SHARED((_NS, _NP), jnp.float32),
        ]

    @functools.partial(
        pl.kernel, mesh=mesh, out_type=tuple(out_type),
        scratch_types=scratch,
        compiler_params=pltpu.CompilerParams(needs_layout_passes=False),
    )
    def k(*refs):
        if with_deg:
            (t_hbm, src_hbm, dst_hbm, z_hbm, z1_hbm,
             p_hbm, pd_hbm, srcv0, srcv1, dstv0, dstv1, rows0, rows1, acc,
             i_sem, g_sem, s_sem, deg_v, dsum_v, tmp_v, sdeg) = refs
        else:
            (t_hbm, src_hbm, dst_hbm, z_hbm,
             p_hbm, srcv0, srcv1, dstv0, dstv1, rows0, rows1, acc,
             i_sem, g_sem, s_sem) = refs
        cid = lax.axis_index("c")
        sid = lax.axis_index("s")
        r0 = sid * _RPT
        bufs = (rows0, rows1)
        srcv = (srcv0, srcv1)
        dstv = (dstv0, dstv1)
        nz = _RPT // _C
        base = (cid * _NS + sid) * _EPW

        def idx(j, p):
            e0 = base + jnp.minimum(j, _NCHUNK - 1) * _C
            return (pltpu.make_async_copy(src_hbm.at[pl.ds(e0, _C)],
                                          srcv[p], i_sem),
                    pltpu.make_async_copy(dst_hbm.at[pl.ds(e0, _C)],
                                          dstv[p], i_sem))

        def gather(p):
            return pltpu.make_async_copy(t_hbm.at[srcv[p]], bufs[p], g_sem)

        def scat(p):
            pltpu.make_async_copy(bufs[p], acc.at[dstv[p]],
                                  s_sem).start(add=True)
            pltpu.make_async_copy(bufs[p], acc.at[dstv[p]], s_sem).wait()

        def deg_add(p):
            if with_deg:
                for i in range(_C // 16):
                    d16 = dstv[p][pl.ds(i * 16, 16)]
                    plsc.addupdate_scatter(
                        deg_v, [d16], jnp.full((16,), 1.0, jnp.float32))

        pltpu.sync_copy(z_hbm.at[pl.ds(0, _C)], rows0)
        for kk in range(nz):
            pltpu.make_async_copy(
                rows0, acc.at[pl.ds(r0 + kk * _C, _C)], s_sem).start()
        if with_deg:
            pltpu.sync_copy(z1_hbm, deg_v)
        for kk in range(nz):
            pltpu.make_async_copy(
                rows0, acc.at[pl.ds(r0 + kk * _C, _C)], s_sem).wait()
        plsc.subcore_barrier()

        for d in idx(0, 0):
            d.start()
        for d in idx(0, 0):
            d.wait()
        gather(0).start()
        for d in idx(1, 1):
            d.start()

        def step(j, p):
            gather(p).wait()
            for d in idx(j + 1, 1 - p):
                d.wait()
            gather(1 - p).start()
            scat(p)
            deg_add(p)
            for d in idx(j + 2, p):
                d.start()

        def body(j2, carry):
            j = j2 * 2
            step(j, 0)
            step(j + 1, 1)
            return carry

        lax.fori_loop(0, (_NCHUNK - 1) // 2, body, 0)
        jt = _NCHUNK - 1
        gather(0).wait()
        for d in idx(jt + 1, 1):
            d.wait()
        scat(0)
        deg_add(0)

        if with_deg:
            pltpu.sync_copy(deg_v, sdeg.at[sid])
        plsc.subcore_barrier()

        o0 = cid * _NP + r0

        def rd(kk):
            return pltpu.make_async_copy(
                acc.at[pl.ds(r0 + kk * _C, _C)], bufs[kk % 2], g_sem)

        def wr(kk):
            return pltpu.make_async_copy(
                bufs[kk % 2], p_hbm.at[pl.ds(o0 + kk * _C, _C)], s_sem)

        rd(0).start()
        for kk in range(nz):
            rd(kk).wait()
            if kk + 1 < nz:
                rd(kk + 1).start()
            wr(kk).start()
            wr(kk).wait()

        if with_deg:
            pltpu.sync_copy(sdeg.at[0, pl.ds(r0, _RPT)], dsum_v)

            def rbody(t, carry):
                pltpu.sync_copy(sdeg.at[t, pl.ds(r0, _RPT)], tmp_v)
                for i in range(_RPT // 16):
                    plsc.addupdate(dsum_v.at[pl.ds(i * 16, 16)],
                                   tmp_v[pl.ds(i * 16, 16)])
                return carry

            lax.fori_loop(1, _NS, rbody, 0)
            pltpu.sync_copy(dsum_v, pd_hbm.at[pl.ds(o0, _RPT)])

    return k


_BR = 1000


def _tc_transform_body(x_ref, w_ref, o_ref):
    o_ref[...] = jnp.dot(x_ref[...], w_ref[...],
                         preferred_element_type=jnp.float32)


def _tc_transform(x, w):
    d_in, d_out = w.shape
    return pl.pallas_call(
        _tc_transform_body,
        grid=(_N // _BR,),
        in_specs=[pl.BlockSpec((_BR, d_in), lambda i: (i, 0)),
                  pl.BlockSpec((d_in, d_out), lambda i: (0, 0))],
        out_specs=pl.BlockSpec((_BR, d_out), lambda i: (i, 0)),
        out_shape=jax.ShapeDtypeStruct((_N, d_out), jnp.float32),
    )(x, w)


def _combine(h, w_self, b, p0, p1, d0, d1):
    inv = 1.0 / jnp.maximum(d0 + d1, 1.0)
    return (jnp.dot(h, w_self, preferred_element_type=jnp.float32)
            + (p0 + p1) * inv + b)


def _tc_mid_body(h_ref, p0_ref, p1_ref, d0_ref, d1_ref, ws_ref, b_ref,
                 wn_ref, hn_ref, tn_ref):
    z = _combine(h_ref[...], ws_ref[...], b_ref[...], p0_ref[...],
                 p1_ref[...], d0_ref[...], d1_ref[...])
    hn = jnp.where(z > 0, z, jnp.exp(jnp.minimum(z, 0.0)) - 1.0)
    hn_ref[...] = hn
    tn_ref[...] = jnp.dot(hn, wn_ref[...], preferred_element_type=jnp.float32)


def _tc_mid(h, p0, p1, d0, d1, w_self, b, w_next):
    d = h.shape[1]
    dn = w_next.shape[1]
    return pl.pallas_call(
        _tc_mid_body,
        grid=(_N // _BR,),
        in_specs=[pl.BlockSpec((_BR, d), lambda i: (i, 0)),
                  pl.BlockSpec((_BR, d), lambda i: (i, 0)),
                  pl.BlockSpec((_BR, d), lambda i: (i, 0)),
                  pl.BlockSpec((_BR, 1), lambda i: (i, 0)),
                  pl.BlockSpec((_BR, 1), lambda i: (i, 0)),
                  pl.BlockSpec((d, d), lambda i: (0, 0)),
                  pl.BlockSpec((1, d), lambda i: (0, 0)),
                  pl.BlockSpec((d, dn), lambda i: (0, 0))],
        out_specs=[pl.BlockSpec((_BR, d), lambda i: (i, 0)),
                   pl.BlockSpec((_BR, dn), lambda i: (i, 0))],
        out_shape=[jax.ShapeDtypeStruct((_N, d), jnp.float32),
                   jax.ShapeDtypeStruct((_N, dn), jnp.float32)],
    )(h, p0, p1, d0, d1, w_self, b.reshape(1, d), w_next)


def _tc_last_body(h_ref, p0_ref, p1_ref, d0_ref, d1_ref, ws_ref, b_ref,
                  o_ref):
    o_ref[...] = _combine(h_ref[...], ws_ref[...], b_ref[...], p0_ref[...],
                          p1_ref[...], d0_ref[...], d1_ref[...])


def _tc_last(h, p0, p1, d0, d1, w_self, b):
    d = h.shape[1]
    dn = w_self.shape[1]
    return pl.pallas_call(
        _tc_last_body,
        grid=(_N // _BR,),
        in_specs=[pl.BlockSpec((_BR, d), lambda i: (i, 0)),
                  pl.BlockSpec((_BR, dn), lambda i: (i, 0)),
                  pl.BlockSpec((_BR, dn), lambda i: (i, 0)),
                  pl.BlockSpec((_BR, 1), lambda i: (i, 0)),
                  pl.BlockSpec((_BR, 1), lambda i: (i, 0)),
                  pl.BlockSpec((d, dn), lambda i: (0, 0)),
                  pl.BlockSpec((1, dn), lambda i: (0, 0))],
        out_specs=pl.BlockSpec((_BR, dn), lambda i: (i, 0)),
        out_shape=jax.ShapeDtypeStruct((_N, dn), jnp.float32),
    )(h, p0, p1, d0, d1, w_self, b.reshape(1, dn))


def kernel(x, edge_index,
           W_self0, W_neigh0, b0,
           W_self1, W_neigh1, b1,
           W_self2, W_neigh2, b2):
    src = edge_index[0].astype(jnp.int32)
    dst = edge_index[1].astype(jnp.int32)
    z128 = jnp.zeros((_NP, 128), jnp.float32)
    z1 = jnp.zeros((_NP,), jnp.float32)

    sc128_deg = _sc_scatter(128, True)
    sc128 = _sc_scatter(128, False)

    t0 = _tc_transform(x, W_neigh0)
    p, pd = sc128_deg(t0, src, dst, z128, z1)
    d0 = pd[:_N, None]
    d1 = pd[_NP:_NP + _N, None]
    h1, t1 = _tc_mid(x, p[:_N], p[_NP:_NP + _N], d0, d1, W_self0, b0,
                     W_neigh1)
    (p,) = sc128(t1, src, dst, z128)
    w2p = jnp.pad(W_neigh2, ((0, 0), (0, 64)))
    h2, t2 = _tc_mid(h1, p[:_N], p[_NP:_NP + _N], d0, d1, W_self1, b1, w2p)
    (p,) = sc128(t2, src, dst, z128)
    return _tc_last(h2, p[:_N, :64], p[_NP:_NP + _N, :64], d0, d1,
                    W_self2, b2)


# speedup vs baseline: 9.5387x; 1.0455x over previous
import functools

import jax
import jax.numpy as jnp
from jax import lax
from jax.experimental import pallas as pl
from jax.experimental.pallas import tpu as pltpu
from jax.experimental.pallas import tpu_sc as plsc

_N = 10000
_E = 320000
_NC = 2
_NS = 16
_NW = _NC * _NS
_EPW = _E // _NW
_C = 80
_NCHUNK = _EPW // _C
_NP = 10240
_RPT = _NP // _NS


def _sc_scatter(d_feat, with_deg):
    mesh = plsc.VectorSubcoreMesh(core_axis_name="c", subcore_axis_name="s")
    out_type = [jax.ShapeDtypeStruct((_NC * _NP, d_feat), jnp.float32)]
    scratch = [
        pltpu.VMEM((_C,), jnp.int32),
        pltpu.VMEM((_C,), jnp.int32),
        pltpu.VMEM((_C,), jnp.int32),
        pltpu.VMEM((_C,), jnp.int32),
        pltpu.VMEM((_C, d_feat), jnp.float32),
        pltpu.VMEM((_C, d_feat), jnp.float32),
        pltpu.VMEM_SHARED((_NP, d_feat), jnp.float32),
        pltpu.SemaphoreType.DMA,
        pltpu.SemaphoreType.DMA,
        pltpu.SemaphoreType.DMA,
    ]
    if with_deg:
        out_type.append(jax.ShapeDtypeStruct((_NC * _NP,), jnp.float32))
        scratch += [
            pltpu.VMEM((_NP,), jnp.float32),
            pltpu.VMEM((_RPT,), jnp.float32),
            pltpu.VMEM((_RPT,), jnp.float32),
            pltpu.VMEM_SHARED((_NS, _NP), jnp.float32),
        ]

    @functools.partial(
        pl.kernel, mesh=mesh, out_type=tuple(out_type),
        scratch_types=scratch,
        compiler_params=pltpu.CompilerParams(needs_layout_passes=False),
    )
    def k(*refs):
        if with_deg:
            (t_hbm, src_hbm, dst_hbm, z_hbm, z1_hbm,
             p_hbm, pd_hbm, srcv0, srcv1, dstv0, dstv1, rows0, rows1, acc,
             i_sem, g_sem, s_sem, deg_v, dsum_v, tmp_v, sdeg) = refs
        else:
            (t_hbm, src_hbm, dst_hbm, z_hbm,
             p_hbm, srcv0, srcv1, dstv0, dstv1, rows0, rows1, acc,
             i_sem, g_sem, s_sem) = refs
        cid = lax.axis_index("c")
        sid = lax.axis_index("s")
        r0 = sid * _RPT
        bufs = (rows0, rows1)
        srcv = (srcv0, srcv1)
        dstv = (dstv0, dstv1)
        nz = _RPT // _C
        base = (cid * _NS + sid) * _EPW

        def idx(j, p):
            e0 = base + jnp.minimum(j, _NCHUNK - 1) * _C
            return (pltpu.make_async_copy(src_hbm.at[pl.ds(e0, _C)],
                                          srcv[p], i_sem),
                    pltpu.make_async_copy(dst_hbm.at[pl.ds(e0, _C)],
                                          dstv[p], i_sem))

        def gather(p):
            return pltpu.make_async_copy(t_hbm.at[srcv[p]], bufs[p], g_sem)

        def scat(p):
            pltpu.make_async_copy(bufs[p], acc.at[dstv[p]],
                                  s_sem).start(add=True)
            pltpu.make_async_copy(bufs[p], acc.at[dstv[p]], s_sem).wait()

        def deg_add(p):
            if with_deg:
                for i in range(_C // 16):
                    d16 = dstv[p][pl.ds(i * 16, 16)]
                    plsc.addupdate_scatter(
                        deg_v, [d16], jnp.full((16,), 1.0, jnp.float32))

        pltpu.sync_copy(z_hbm, rows0)
        for kk in range(nz):
            pltpu.make_async_copy(
                rows0, acc.at[pl.ds(r0 + kk * _C, _C)], s_sem).start()
        if with_deg:
            pltpu.sync_copy(z1_hbm, deg_v)
        for kk in range(nz):
            pltpu.make_async_copy(
                rows0, acc.at[pl.ds(r0 + kk * _C, _C)], s_sem).wait()
        plsc.subcore_barrier()

        for d in idx(0, 0):
            d.start()
        for d in idx(0, 0):
            d.wait()
        gather(0).start()
        for d in idx(1, 1):
            d.start()

        def step(j, p):
            gather(p).wait()
            for d in idx(j + 1, 1 - p):
                d.wait()
            gather(1 - p).start()
            scat(p)
            deg_add(p)
            for d in idx(j + 2, p):
                d.start()

        def body(j2, carry):
            j = j2 * 2
            step(j, 0)
            step(j + 1, 1)
            return carry

        lax.fori_loop(0, (_NCHUNK - 1) // 2, body, 0)
        jt = _NCHUNK - 1
        gather(0).wait()
        for d in idx(jt + 1, 1):
            d.wait()
        scat(0)
        deg_add(0)

        if with_deg:
            pltpu.sync_copy(deg_v, sdeg.at[sid])
        plsc.subcore_barrier()

        o0 = cid * _NP + r0

        def rd(kk):
            return pltpu.make_async_copy(
                acc.at[pl.ds(r0 + kk * _C, _C)], bufs[kk % 2], g_sem)

        def wr(kk):
            return pltpu.make_async_copy(
                bufs[kk % 2], p_hbm.at[pl.ds(o0 + kk * _C, _C)], s_sem)

        rd(0).start()
        for kk in range(nz):
            rd(kk).wait()
            if kk + 1 < nz:
                rd(kk + 1).start()
            wr(kk).start()
            wr(kk).wait()

        if with_deg:
            pltpu.sync_copy(sdeg.at[0, pl.ds(r0, _RPT)], dsum_v)

            def rbody(t, carry):
                pltpu.sync_copy(sdeg.at[t, pl.ds(r0, _RPT)], tmp_v)
                for i in range(_RPT // 16):
                    plsc.addupdate(dsum_v.at[pl.ds(i * 16, 16)],
                                   tmp_v[pl.ds(i * 16, 16)])
                return carry

            lax.fori_loop(1, _NS, rbody, 0)
            pltpu.sync_copy(dsum_v, pd_hbm.at[pl.ds(o0, _RPT)])

    return k


_BR = 1024
_NB = _NP // _BR


def _tc_transform_body(x_ref, w_ref, o_ref):
    o_ref[...] = jnp.dot(x_ref[...], w_ref[...],
                         preferred_element_type=jnp.float32)


def _tc_transform(x, w):
    d_in, d_out = w.shape
    return pl.pallas_call(
        _tc_transform_body,
        grid=(_NB,),
        in_specs=[pl.BlockSpec((_BR, d_in), lambda i: (i, 0)),
                  pl.BlockSpec((d_in, d_out), lambda i: (0, 0))],
        out_specs=pl.BlockSpec((_BR, d_out), lambda i: (i, 0)),
        out_shape=jax.ShapeDtypeStruct((_NP, d_out), jnp.float32),
    )(x, w)


def _combine(h, w_self, b, p0, p1, d0, d1):
    inv = 1.0 / jnp.maximum(d0 + d1, 1.0)
    return (jnp.dot(h, w_self, preferred_element_type=jnp.float32)
            + (p0 + p1) * inv + b)


def _tc_mid_body(h_ref, p0_ref, p1_ref, d0_ref, d1_ref, ws_ref, b_ref,
                 wn_ref, hn_ref, tn_ref):
    z = _combine(h_ref[...], ws_ref[...], b_ref[...], p0_ref[...],
                 p1_ref[...], d0_ref[...], d1_ref[...])
    hn = jnp.where(z > 0, z, jnp.exp(jnp.minimum(z, 0.0)) - 1.0)
    hn_ref[...] = hn
    tn_ref[...] = jnp.dot(hn, wn_ref[...], preferred_element_type=jnp.float32)


def _tc_mid(h, p, pd, w_self, b, w_next):
    d = h.shape[1]
    dn = w_next.shape[1]
    return pl.pallas_call(
        _tc_mid_body,
        grid=(_NB,),
        in_specs=[pl.BlockSpec((_BR, d), lambda i: (i, 0)),
                  pl.BlockSpec((_BR, d), lambda i: (i, 0)),
                  pl.BlockSpec((_BR, d), lambda i: (_NB + i, 0)),
                  pl.BlockSpec((_BR, 1), lambda i: (i, 0)),
                  pl.BlockSpec((_BR, 1), lambda i: (_NB + i, 0)),
                  pl.BlockSpec((d, d), lambda i: (0, 0)),
                  pl.BlockSpec((1, d), lambda i: (0, 0)),
                  pl.BlockSpec((d, dn), lambda i: (0, 0))],
        out_specs=[pl.BlockSpec((_BR, d), lambda i: (i, 0)),
                   pl.BlockSpec((_BR, dn), lambda i: (i, 0))],
        out_shape=[jax.ShapeDtypeStruct((_NP, d), jnp.float32),
                   jax.ShapeDtypeStruct((_NP, dn), jnp.float32)],
    )(h, p, p, pd, pd, w_self, b.reshape(1, d), w_next)


def _tc_last_body(h_ref, p0_ref, p1_ref, d0_ref, d1_ref, ws_ref, b_ref,
                  o_ref):
    o_ref[...] = _combine(h_ref[...], ws_ref[...], b_ref[...],
                          p0_ref[..., :64], p1_ref[..., :64],
                          d0_ref[...], d1_ref[...])


def _tc_last(h, p, pd, w_self, b):
    d = h.shape[1]
    dn = w_self.shape[1]
    return pl.pallas_call(
        _tc_last_body,
        grid=(_NB,),
        in_specs=[pl.BlockSpec((_BR, d), lambda i: (i, 0)),
                  pl.BlockSpec((_BR, d), lambda i: (i, 0)),
                  pl.BlockSpec((_BR, d), lambda i: (_NB + i, 0)),
                  pl.BlockSpec((_BR, 1), lambda i: (i, 0)),
                  pl.BlockSpec((_BR, 1), lambda i: (_NB + i, 0)),
                  pl.BlockSpec((d, dn), lambda i: (0, 0)),
                  pl.BlockSpec((1, dn), lambda i: (0, 0))],
        out_specs=pl.BlockSpec((_BR, dn), lambda i: (i, 0)),
        out_shape=jax.ShapeDtypeStruct((_NP, dn), jnp.float32),
    )(h, p, p, pd, pd, w_self, b.reshape(1, dn))


def kernel(x, edge_index,
           W_self0, W_neigh0, b0,
           W_self1, W_neigh1, b1,
           W_self2, W_neigh2, b2):
    src = edge_index[0].astype(jnp.int32)
    dst = edge_index[1].astype(jnp.int32)
    z128 = jnp.zeros((_C, 128), jnp.float32)
    z1 = jnp.zeros((_NP,), jnp.float32)
    xp = jnp.pad(x, ((0, _NP - _N), (0, 0)))

    sc128_deg = _sc_scatter(128, True)
    sc128 = _sc_scatter(128, False)

    t0 = _tc_transform(xp, W_neigh0)
    p, pd = sc128_deg(t0, src, dst, z128, z1)
    pd2 = pd[:, None]
    h1, t1 = _tc_mid(xp, p, pd2, W_self0, b0, W_neigh1)
    (p,) = sc128(t1, src, dst, z128)
    w2p = jnp.pad(W_neigh2, ((0, 0), (0, 64)))
    h2, t2 = _tc_mid(h1, p, pd2, W_self1, b1, w2p)
    (p,) = sc128(t2, src, dst, z128)
    return _tc_last(h2, p, pd2, W_self2, b2)[:_N]


# speedup vs baseline: 11.4211x; 1.1973x over previous
import functools

import jax
import jax.numpy as jnp
from jax import lax
from jax.experimental import pallas as pl
from jax.experimental.pallas import tpu as pltpu
from jax.experimental.pallas import tpu_sc as plsc

_N = 10000
_E = 320000
_NC = 2
_NS = 16
_NW = _NC * _NS
_C = 128
_NCHB = _E // _C
_NP = 10240
_RPT = _NP // _NS


def _sc_scatter(d_feat, with_deg):
    mesh = plsc.VectorSubcoreMesh(core_axis_name="c", subcore_axis_name="s")
    out_type = [jax.ShapeDtypeStruct((_NC * _NP, d_feat), jnp.float32)]
    scratch = [
        pltpu.VMEM((_C,), jnp.int32),
        pltpu.VMEM((_C,), jnp.int32),
        pltpu.VMEM((_C,), jnp.int32),
        pltpu.VMEM((_C,), jnp.int32),
        pltpu.VMEM((_C, d_feat), jnp.float32),
        pltpu.VMEM((_C, d_feat), jnp.float32),
        pltpu.VMEM_SHARED((_NP, d_feat), jnp.float32),
        pltpu.SemaphoreType.DMA,
        pltpu.SemaphoreType.DMA,
        pltpu.SemaphoreType.DMA,
    ]
    if with_deg:
        out_type.append(jax.ShapeDtypeStruct((_NW, _NP), jnp.float32))
        scratch += [
            pltpu.VMEM((_NP,), jnp.float32),
        ]

    @functools.partial(
        pl.kernel, mesh=mesh, out_type=tuple(out_type),
        scratch_types=scratch,
        compiler_params=pltpu.CompilerParams(needs_layout_passes=False),
    )
    def k(*refs):
        if with_deg:
            (t_hbm, src_hbm, dst_hbm, z_hbm, z1_hbm,
             p_hbm, pd_hbm, srcv0, srcv1, dstv0, dstv1, rows0, rows1, acc,
             i_sem, g_sem, s_sem, deg_v) = refs
        else:
            (t_hbm, src_hbm, dst_hbm, z_hbm,
             p_hbm, srcv0, srcv1, dstv0, dstv1, rows0, rows1, acc,
             i_sem, g_sem, s_sem) = refs
        cid = lax.axis_index("c")
        sid = lax.axis_index("s")
        r0 = sid * _RPT
        bufs = (rows0, rows1)
        srcv = (srcv0, srcv1)
        dstv = (dstv0, dstv1)
        nz = _RPT // _C
        w = cid * _NS + sid
        n_b = 79 - 2 * (w >= 18).astype(jnp.int32)
        base = (77 * w + 2 * jnp.minimum(w, 18)) * _C

        def idx(j, p):
            e0 = base + jnp.minimum(j, n_b - 1) * _C
            return (pltpu.make_async_copy(src_hbm.at[pl.ds(e0, _C)],
                                          srcv[p], i_sem),
                    pltpu.make_async_copy(dst_hbm.at[pl.ds(e0, _C)],
                                          dstv[p], i_sem))

        def gather(p):
            return pltpu.make_async_copy(t_hbm.at[srcv[p]], bufs[p], g_sem)

        def scat(p):
            pltpu.make_async_copy(bufs[p], acc.at[dstv[p]],
                                  s_sem).start(add=True)
            pltpu.make_async_copy(bufs[p], acc.at[dstv[p]], s_sem).wait()

        def deg_add(p):
            if with_deg:
                for i in range(_C // 16):
                    d16 = dstv[p][pl.ds(i * 16, 16)]
                    plsc.addupdate_scatter(
                        deg_v, [d16], jnp.full((16,), 1.0, jnp.float32))

        pltpu.sync_copy(z_hbm, rows0)
        for kk in range(nz):
            pltpu.make_async_copy(
                rows0, acc.at[pl.ds(r0 + kk * _C, _C)], s_sem).start()
        if with_deg:
            pltpu.sync_copy(z1_hbm, deg_v)
        for kk in range(nz):
            pltpu.make_async_copy(
                rows0, acc.at[pl.ds(r0 + kk * _C, _C)], s_sem).wait()
        plsc.subcore_barrier()

        for d in idx(0, 0):
            d.start()
        for d in idx(0, 0):
            d.wait()
        gather(0).start()
        for d in idx(1, 1):
            d.start()

        def step(j, p):
            gather(p).wait()
            for d in idx(j + 1, 1 - p):
                d.wait()
            gather(1 - p).start()
            scat(p)
            deg_add(p)
            for d in idx(j + 2, p):
                d.start()

        def body(j2, carry):
            j = j2 * 2
            step(j, 0)
            step(j + 1, 1)
            return carry

        lax.fori_loop(0, (n_b - 1) // 2, body, 0)
        jt = n_b - 1
        gather(0).wait()
        for d in idx(jt + 1, 1):
            d.wait()
        scat(0)
        deg_add(0)

        if with_deg:
            pltpu.sync_copy(deg_v, pd_hbm.at[w])
        plsc.subcore_barrier()

        o0 = cid * _NP + r0

        def rd(kk):
            return pltpu.make_async_copy(
                acc.at[pl.ds(r0 + kk * _C, _C)], bufs[kk % 2], g_sem)

        def wr(kk):
            return pltpu.make_async_copy(
                bufs[kk % 2], p_hbm.at[pl.ds(o0 + kk * _C, _C)], s_sem)

        rd(0).start()
        for kk in range(nz):
            rd(kk).wait()
            if kk + 1 < nz:
                rd(kk + 1).start()
            wr(kk).start()
            wr(kk).wait()

    return k


_BR = 1024
_NB = _NP // _BR


def _tc_transform_body(x_ref, w_ref, o_ref):
    o_ref[...] = jnp.dot(x_ref[...], w_ref[...],
                         preferred_element_type=jnp.float32)


def _tc_transform(x, w):
    d_in, d_out = w.shape
    return pl.pallas_call(
        _tc_transform_body,
        grid=(_NB,),
        in_specs=[pl.BlockSpec((_BR, d_in), lambda i: (i, 0)),
                  pl.BlockSpec((d_in, d_out), lambda i: (0, 0))],
        out_specs=pl.BlockSpec((_BR, d_out), lambda i: (i, 0)),
        out_shape=jax.ShapeDtypeStruct((_NP, d_out), jnp.float32),
    )(x, w)


def _combine(h, w_self, b, p0, p1, pd):
    deg = jnp.sum(pd, axis=0)[:, None]
    inv = 1.0 / jnp.maximum(deg, 1.0)
    return (jnp.dot(h, w_self, preferred_element_type=jnp.float32)
            + (p0 + p1) * inv + b)


def _tc_mid_body(h_ref, p0_ref, p1_ref, pd_ref, ws_ref, b_ref,
                 wn_ref, hn_ref, tn_ref):
    z = _combine(h_ref[...], ws_ref[...], b_ref[...], p0_ref[...],
                 p1_ref[...], pd_ref[...])
    hn = jnp.where(z > 0, z, jnp.exp(jnp.minimum(z, 0.0)) - 1.0)
    hn_ref[...] = hn
    tn_ref[...] = jnp.dot(hn, wn_ref[...], preferred_element_type=jnp.float32)


def _tc_mid(h, p, pd, w_self, b, w_next):
    d = h.shape[1]
    dn = w_next.shape[1]
    return pl.pallas_call(
        _tc_mid_body,
        grid=(_NB,),
        in_specs=[pl.BlockSpec((_BR, d), lambda i: (i, 0)),
                  pl.BlockSpec((_BR, d), lambda i: (i, 0)),
                  pl.BlockSpec((_BR, d), lambda i: (_NB + i, 0)),
                  pl.BlockSpec((_NW, _BR), lambda i: (0, i)),
                  pl.BlockSpec((d, d), lambda i: (0, 0)),
                  pl.BlockSpec((1, d), lambda i: (0, 0)),
                  pl.BlockSpec((d, dn), lambda i: (0, 0))],
        out_specs=[pl.BlockSpec((_BR, d), lambda i: (i, 0)),
                   pl.BlockSpec((_BR, dn), lambda i: (i, 0))],
        out_shape=[jax.ShapeDtypeStruct((_NP, d), jnp.float32),
                   jax.ShapeDtypeStruct((_NP, dn), jnp.float32)],
    )(h, p, p, pd, w_self, b.reshape(1, d), w_next)


def _tc_last_body(h_ref, p0_ref, p1_ref, pd_ref, ws_ref, b_ref,
                  o_ref):
    o_ref[...] = _combine(h_ref[...], ws_ref[...], b_ref[...],
                          p0_ref[..., :64], p1_ref[..., :64],
                          pd_ref[...])


def _tc_last(h, p, pd, w_self, b):
    d = h.shape[1]
    dn = w_self.shape[1]
    return pl.pallas_call(
        _tc_last_body,
        grid=(_NB,),
        in_specs=[pl.BlockSpec((_BR, d), lambda i: (i, 0)),
                  pl.BlockSpec((_BR, d), lambda i: (i, 0)),
                  pl.BlockSpec((_BR, d), lambda i: (_NB + i, 0)),
                  pl.BlockSpec((_NW, _BR), lambda i: (0, i)),
                  pl.BlockSpec((d, dn), lambda i: (0, 0)),
                  pl.BlockSpec((1, dn), lambda i: (0, 0))],
        out_specs=pl.BlockSpec((_BR, dn), lambda i: (i, 0)),
        out_shape=jax.ShapeDtypeStruct((_NP, dn), jnp.float32),
    )(h, p, p, pd, w_self, b.reshape(1, dn))


def kernel(x, edge_index,
           W_self0, W_neigh0, b0,
           W_self1, W_neigh1, b1,
           W_self2, W_neigh2, b2):
    src = edge_index[0].astype(jnp.int32)
    dst = edge_index[1].astype(jnp.int32)
    z128 = jnp.zeros((_C, 128), jnp.float32)
    z1 = jnp.zeros((_NP,), jnp.float32)
    xp = jnp.pad(x, ((0, _NP - _N), (0, 0)))

    sc128_deg = _sc_scatter(128, True)
    sc128 = _sc_scatter(128, False)

    t0 = _tc_transform(xp, W_neigh0)
    p, pd = sc128_deg(t0, src, dst, z128, z1)
    h1, t1 = _tc_mid(xp, p, pd, W_self0, b0, W_neigh1)
    (p,) = sc128(t1, src, dst, z128)
    w2p = jnp.pad(W_neigh2, ((0, 0), (0, 64)))
    h2, t2 = _tc_mid(h1, p, pd, W_self1, b1, w2p)
    (p,) = sc128(t2, src, dst, z128)
    return _tc_last(h2, p, pd, W_self2, b2)[:_N]
